# baseline (speedup 1.0000x reference)
import functools

import jax
import jax.numpy as jnp
from jax import lax
from jax.experimental import pallas as pl
from jax.experimental.pallas import tpu as pltpu
from jax.experimental.pallas import tpu_sc as plsc

H = 768
HV = H // 16
C = 32
CP = C + 16
NR = 3
NW = 32
EPS = 1e-12


def _rsqrt_scalar(x):
    xb = lax.bitcast_convert_type(x, jnp.int32)
    yb = jnp.int32(0x5F3759DF) - lax.shift_right_arithmetic(xb, jnp.int32(1))
    y = lax.bitcast_convert_type(yb, jnp.float32)
    half = x * 0.5
    for _ in range(3):
        y = y * (1.5 - half * y * y)
    return y


def kernel(input_ids, token_type_ids, word_emb, pos_emb, type_emb, gamma, beta):
    B, S = input_ids.shape
    PCHUNKS = S // C
    GPB = NW // PCHUNKS
    NB = B // GPB

    mesh = plsc.VectorSubcoreMesh(core_axis_name="c", subcore_axis_name="s")

    @functools.partial(
        pl.kernel,
        out_type=jax.ShapeDtypeStruct((B, S, H), jnp.float32),
        mesh=mesh,
        scratch_types=[
            pltpu.VMEM((NR * C,), jnp.int32),
            pltpu.VMEM((NR * CP,), jnp.int32),
            pltpu.VMEM((2 * C, H), jnp.float32),
            pltpu.VMEM((2, H), jnp.float32),
            pltpu.VMEM((H,), jnp.float32),
            pltpu.VMEM((H,), jnp.float32),
            pltpu.VMEM((NR * C, H), jnp.float32),
            pltpu.VMEM((64,), jnp.float32),
            pltpu.SemaphoreType.DMA((NR,)),
            pltpu.SemaphoreType.DMA((NR,)),
        ],
    )
    def sc_kernel(ids_h, tt_h, wemb_h, pos_h, type_h, gam_h, bet_h, out_h,
                  idx_v, ttv, comb2, ty_v, gam_v, bet_v, rows, red_v,
                  gsem, osem):
        cid = lax.axis_index("c")
        sid = lax.axis_index("s")
        w = sid * 2 + cid
        pc = w // GPB
        bg = w % GPB
        pos0 = pc * C
        b0 = bg * NB

        red_v[pl.ds(16, 16)] = jnp.zeros((16,), jnp.float32)
        red_v[pl.ds(48, 16)] = jnp.zeros((16,), jnp.float32)
        pltpu.sync_copy(pos_h.at[pl.ds(pos0, C)], comb2.at[pl.ds(0, C)])
        pltpu.sync_copy(pos_h.at[pl.ds(pos0, C)], comb2.at[pl.ds(C, C)])
        pltpu.sync_copy(type_h, ty_v)
        pltpu.sync_copy(gam_h, gam_v)
        pltpu.sync_copy(bet_h, bet_v)

        def addty(t, carry):
            for h in range(HV):
                sl = pl.ds(h * 16, 16)
                comb2[t, sl] = comb2[t, sl] + ty_v[0, sl]
                comb2[C + t, sl] = comb2[C + t, sl] + ty_v[1, sl]
            return carry

        lax.fori_loop(0, C, addty, 0)

        def start_fetch(j):
            p = lax.rem(j, NR)
            b = b0 + j
            pltpu.sync_copy(ids_h.at[b, pl.ds(pos0, C)],
                            idx_v.at[pl.ds(p * C, C)])
            pltpu.sync_copy(tt_h.at[b, pl.ds(pos0, C)],
                            ttv.at[pl.ds(p * CP, C)])
            pltpu.async_copy(wemb_h.at[idx_v.at[pl.ds(p * C, C)]],
                             rows.at[pl.ds(p * C, C)], gsem.at[p])

        def wait_gather(p):
            pltpu.make_async_copy(
                wemb_h.at[idx_v.at[pl.ds(p * C, C)]],
                rows.at[pl.ds(p * C, C)], gsem.at[p]).wait()

        def out_descr(j, p):
            b = b0 + j
            return pltpu.make_async_copy(
                rows.at[pl.ds(p * C, C)],
                out_h.at[b, pl.ds(pos0, C)], osem.at[p])

        def tok_body(t, pC):
            r = pC + t
            tts = ttv[pl.ds((pC // C) * CP + t, 16)][0]
            cb = tts * C + t
            s1 = [jnp.zeros((16,), jnp.float32) for _ in range(4)]
            s2 = [jnp.zeros((16,), jnp.float32) for _ in range(4)]
            for h in range(HV):
                sl = pl.ds(h * 16, 16)
                x = rows[r, sl] + comb2[cb, sl]
                rows[r, sl] = x
                k = h & 3
                s1[k] = s1[k] + x
                s2[k] = s2[k] + x * x
            v1 = (s1[0] + s1[1]) + (s1[2] + s1[3])
            v2 = (s2[0] + s2[1]) + (s2[2] + s2[3])
            red_v[pl.ds(0, 16)] = v1
            red_v[pl.ds(32, 16)] = v2
            for k in (8, 4, 2, 1):
                red_v[pl.ds(0, 16)] = red_v[pl.ds(0, 16)] + red_v[pl.ds(k, 16)]
                red_v[pl.ds(32, 16)] = (
                    red_v[pl.ds(32, 16)] + red_v[pl.ds(32 + k, 16)]
                )
            m_s = red_v[pl.ds(0, 16)][0] * (1.0 / H)
            ex2_s = red_v[pl.ds(32, 16)][0] * (1.0 / H)
            inv_s = _rsqrt_scalar(ex2_s - m_s * m_s + EPS)
            m = jnp.full((16,), m_s)
            inv = jnp.full((16,), inv_s)
            for h in range(HV):
                sl = pl.ds(h * 16, 16)
                y = (rows[r, sl] - m) * inv
                rows[r, sl] = y * gam_v[sl] + bet_v[sl]
            return pC

        start_fetch(jnp.int32(0))

        def batch_body(j, carry):
            @pl.when(j + 1 < NB)
            def _():
                @pl.when(j >= NR - 1)
                def _():
                    q = lax.rem(j + 1, NR)
                    out_descr(j + 1 - NR, q).wait()
                start_fetch(j + 1)

            p = lax.rem(j, NR)
            wait_gather(p)
            lax.fori_loop(0, C, tok_body, p * C)
            out_descr(j, p).start()
            return carry

        lax.fori_loop(0, NB, batch_body, 0)
        for jj in range(NB - NR, NB):
            out_descr(jnp.int32(jj), jnp.int32(jj % NR)).wait()

    return sc_kernel(input_ids, token_type_ids, word_emb, pos_emb, type_emb,
                     gamma, beta)


# speedup vs baseline: 3.5377x; 3.5377x over previous
import functools

import jax
import jax.numpy as jnp
from jax import lax
from jax.experimental import pallas as pl
from jax.experimental.pallas import tpu as pltpu
from jax.experimental.pallas import tpu_sc as plsc

H = 768
NW = 32
CG = 64
NR = 2
EPS = 1e-12


def _sc_gather(ids_flat, word_emb):
    TOK = ids_flat.shape[0]
    TPW = TOK // NW
    NIT = TPW // CG

    mesh = plsc.VectorSubcoreMesh(core_axis_name="c", subcore_axis_name="s")

    @functools.partial(
        pl.kernel,
        out_type=jax.ShapeDtypeStruct((TOK, H), jnp.float32),
        mesh=mesh,
        scratch_types=[
            pltpu.VMEM((TPW,), jnp.int32),
            pltpu.VMEM((NR * CG, H), jnp.float32),
            pltpu.SemaphoreType.DMA((NR,)),
            pltpu.SemaphoreType.DMA((NR,)),
        ],
    )
    def gather_k(ids_h, wemb_h, st_h, idx_v, rows, gsem, osem):
        w = lax.axis_index("s") * 2 + lax.axis_index("c")
        t0 = w * TPW
        pltpu.sync_copy(ids_h.at[pl.ds(t0, TPW)], idx_v)

        def gdesc(j):
            p = lax.rem(j, NR)
            return pltpu.make_async_copy(
                wemb_h.at[idx_v.at[pl.ds(j * CG, CG)]],
                rows.at[pl.ds(p * CG, CG)], gsem.at[p])

        def odesc(j):
            p = lax.rem(j, NR)
            return pltpu.make_async_copy(
                rows.at[pl.ds(p * CG, CG)],
                st_h.at[pl.ds(t0 + j * CG, CG)], osem.at[p])

        gdesc(jnp.int32(0)).start()

        def body(j, c):
            @pl.when(j + 1 < NIT)
            def _():
                @pl.when(j >= NR - 1)
                def _():
                    odesc(j + 1 - NR).wait()
                gdesc(j + 1).start()

            gdesc(j).wait()
            odesc(j).start()
            return c

        lax.fori_loop(0, NIT, body, 0)
        for jj in range(NIT - NR, NIT):
            odesc(jnp.int32(jj)).wait()

    return gather_k(ids_flat, word_emb)


def _tc_add_ln(staged, ttf3, pos_emb, type_emb, gamma2, beta2, B, S):
    def body(st_ref, tt_ref, pos_ref, ty_ref, gam_ref, bet_ref, o_ref):
        x = st_ref[...] + pos_ref[...]
        t0 = ty_ref[0, :]
        d = ty_ref[1, :] - t0
        tt = tt_ref[0, 0, :]
        x = x + t0[None, :] + tt[:, None] * d[None, :]
        m = jnp.mean(x, axis=-1, keepdims=True)
        xc = x - m
        var = jnp.mean(xc * xc, axis=-1, keepdims=True)
        inv = lax.rsqrt(var + EPS)
        o_ref[...] = xc * inv * gam_ref[...] + bet_ref[...]

    return pl.pallas_call(
        body,
        grid=(B,),
        in_specs=[
            pl.BlockSpec((S, H), lambda i: (i, 0)),
            pl.BlockSpec((1, 1, S), lambda i: (i, 0, 0)),
            pl.BlockSpec((S, H), lambda i: (0, 0)),
            pl.BlockSpec((2, H), lambda i: (0, 0)),
            pl.BlockSpec((1, H), lambda i: (0, 0)),
            pl.BlockSpec((1, H), lambda i: (0, 0)),
        ],
        out_specs=pl.BlockSpec((S, H), lambda i: (i, 0)),
        out_shape=jax.ShapeDtypeStruct((B * S, H), jnp.float32),
        compiler_params=pltpu.CompilerParams(
            dimension_semantics=("arbitrary",)),
    )(staged, ttf3, pos_emb, type_emb, gamma2, beta2)


def kernel(input_ids, token_type_ids, word_emb, pos_emb, type_emb, gamma, beta):
    B, S = input_ids.shape
    staged = _sc_gather(input_ids.reshape(-1), word_emb)
    ttf3 = token_type_ids.astype(jnp.float32).reshape(B, 1, S)
    out = _tc_add_ln(staged, ttf3, pos_emb, type_emb,
                     gamma.reshape(1, H), beta.reshape(1, H), B, S)
    return out.reshape(B, S, H)
